# Initial kernel scaffold; baseline (speedup 1.0000x reference)
import jax
import jax.numpy as jnp
from jax.experimental import pallas as pl


def kernel(H, W, relative_position_bias_table):
    raise NotImplementedError("write your pallas kernel here")


# speedup vs baseline: 34.8764x; 34.8764x over previous
import jax
import jax.numpy as jnp
from jax import lax
from jax.experimental import pallas as pl
from jax.experimental.pallas import tpu as pltpu
from jax.experimental.pallas import tpu_sc as plsc


def _sc_expand(c_sh):
    nh = 16
    n = 32

    def body(c_hbm, out_hbm, s_v, sem):
        cid = lax.axis_index("c")
        sid = lax.axis_index("s")
        wid = sid * 2 + cid
        h = wid // 2
        half = wid % 2
        builds = [
            pltpu.async_copy(
                c_hbm.at[(31 - pw) % 8, h, :, pl.ds((31 - pw) - (31 - pw) % 8, n)],
                s_v.at[pw],
                sem,
            )
            for pw in range(n)
        ]
        for b in builds:
            b.wait()
        bands = []
        for i in range(n // 2):
            ph = half * (n // 2) + i
            bands.append(
                pltpu.async_copy(
                    s_v.at[:, pl.ds(31 - ph, n), :], out_hbm.at[h, ph], sem
                )
            )
        for b in bands:
            b.wait()

    run = pl.kernel(
        body,
        out_type=jax.ShapeDtypeStruct((nh, n, n, n, n), jnp.float32),
        mesh=plsc.VectorSubcoreMesh(core_axis_name="c", subcore_axis_name="s"),
        scratch_types=[
            pltpu.VMEM((n, 63, n), jnp.float32),
            pltpu.SemaphoreType.DMA,
        ],
        compiler_params=pltpu.CompilerParams(use_tc_tiling_on_sc=False),
    )
    return run(c_sh)


def kernel(H, W, relative_position_bias_table):
    table = relative_position_bias_table
    nh = table.shape[1]
    side = int(round(table.shape[0] ** 0.5))
    n = (side + 1) // 2
    off = (jnp.asarray(H, jnp.int32) - n) + (jnp.asarray(W, jnp.int32) - n)
    table = jnp.roll(table, -off, axis=0)
    c = jnp.transpose(table.reshape(side, side, nh)[::-1, ::-1, :], (2, 0, 1))
    c_wide = jnp.pad(c, ((0, 0), (0, 0), (0, 72 - side)))
    c_sh = jnp.stack([c_wide[:, :, s:s + 64] for s in range(8)])
    out = _sc_expand(c_sh)
    return out.reshape(1, nh, n * n, n * n)
